# baseline (speedup 1.0000x reference)
import functools

import jax
import jax.numpy as jnp
from jax import lax
from jax.experimental import pallas as pl
from jax.experimental.pallas import tpu as pltpu
from jax.experimental.pallas import tpu_sc as plsc

VOCAB = 1000000
MAXLEN = 200
EMBED_DIM = 64
BATCH = 4096

NUM_CORES = 2
NUM_SUBCORES = 16
LANES = 16
NW = NUM_CORES * NUM_SUBCORES
BCH = BATCH // NW
BLKS = BCH // LANES
NBUF = 2
NGROUPS = MAXLEN // NBUF


def _make_kernel():
    mesh = plsc.VectorSubcoreMesh(core_axis_name="c", subcore_axis_name="s")

    @functools.partial(
        pl.kernel,
        out_type=jax.ShapeDtypeStruct((MAXLEN, EMBED_DIM, BATCH), jnp.float32),
        mesh=mesh,
        scratch_types=[
            pltpu.VMEM((EMBED_DIM, MAXLEN), jnp.float32),
            pltpu.VMEM((MAXLEN, BCH), jnp.int32),
            pltpu.VMEM((NBUF, BCH, EMBED_DIM), jnp.float32),
            pltpu.VMEM((NBUF, EMBED_DIM, BCH), jnp.float32),
            pltpu.SemaphoreType.DMA,
            pltpu.SemaphoreType.DMA,
            pltpu.SemaphoreType.DMA,
            pltpu.SemaphoreType.DMA,
        ],
        compiler_params=pltpu.CompilerParams(use_tc_tiling_on_sc=False,
                                             needs_layout_passes=False),
    )
    def tok_pos_embed(x_hbm, tok_hbm, pos_hbm, out_hbm,
                      pos_v, idx_v, gbuf, tbuf, g0, g1, o0, o1):
        wid = lax.axis_index("s") * NUM_CORES + lax.axis_index("c")
        b0 = wid * BCH
        gsem = (g0, g1)
        osem = (o0, o1)
        pltpu.sync_copy(pos_hbm, pos_v)
        pltpu.sync_copy(x_hbm.at[:, pl.ds(b0, BCH)], idx_v)

        def start_gather(l, bb):
            pltpu.async_copy(tok_hbm.at[idx_v.at[l]], gbuf.at[bb], gsem[bb])

        for bb in range(NBUF):
            start_gather(bb, bb)

        row_ids = [lax.iota(jnp.int32, LANES) + blk * LANES
                   for blk in range(BLKS)]

        def group_body(g, carry):
            for bb in range(NBUF):
                l = g * NBUF + bb
                pltpu.make_async_copy(
                    tok_hbm.at[idx_v.at[l]], gbuf.at[bb], gsem[bb]).wait()

                @pl.when(g >= 1)
                def _wait_prev_out():
                    pltpu.make_async_copy(
                        tbuf.at[bb], out_hbm.at[0, :, pl.ds(b0, BCH)],
                        osem[bb]).wait()

                l_splat = jnp.full((LANES,), l, jnp.int32)

                def per_dim(d, cr):
                    dvec = jnp.full((LANES,), d, jnp.int32)
                    pos_vec = plsc.load_gather(pos_v, [dvec, l_splat])
                    for blk in range(BLKS):
                        v = plsc.load_gather(gbuf.at[bb], [row_ids[blk], dvec])
                        tbuf[bb, d, pl.ds(blk * LANES, LANES)] = v + pos_vec
                    return cr

                lax.fori_loop(0, EMBED_DIM, per_dim, 0, unroll=2)

                @pl.when(g < NGROUPS - 1)
                def _next_gather():
                    start_gather(l + NBUF, bb)

                pltpu.async_copy(
                    tbuf.at[bb], out_hbm.at[l, :, pl.ds(b0, BCH)], osem[bb])
            return carry

        lax.fori_loop(0, NGROUPS, group_body, 0)
        for bb in range(NBUF):
            pltpu.make_async_copy(
                tbuf.at[bb], out_hbm.at[0, :, pl.ds(b0, BCH)], osem[bb]).wait()

    return tok_pos_embed


_kernel_call = _make_kernel()


def kernel(x, token_table, pos_table):
    x_t = jnp.transpose(x.astype(jnp.int32), (1, 0))
    pos_t = jnp.transpose(pos_table, (1, 0))
    out = _kernel_call(x_t, token_table, pos_t)
    return jnp.transpose(out, (2, 0, 1))


# speedup vs baseline: 1.5916x; 1.5916x over previous
import functools

import jax
import jax.numpy as jnp
from jax import lax
from jax.experimental import pallas as pl
from jax.experimental.pallas import tpu as pltpu
from jax.experimental.pallas import tpu_sc as plsc

VOCAB = 1000000
MAXLEN = 200
EMBED_DIM = 64
BATCH = 4096

NUM_CORES = 2
NUM_SUBCORES = 16
LANES = 16
NW = NUM_CORES * NUM_SUBCORES
BCH = BATCH // NW
DQ = EMBED_DIM // LANES
PITCH = BCH + 1
NBUF = 4
NGROUPS = MAXLEN // NBUF


def _make_kernel():
    mesh = plsc.VectorSubcoreMesh(core_axis_name="c", subcore_axis_name="s")

    @functools.partial(
        pl.kernel,
        out_type=jax.ShapeDtypeStruct((MAXLEN, EMBED_DIM, BATCH), jnp.float32),
        mesh=mesh,
        scratch_types=[
            pltpu.VMEM((EMBED_DIM, MAXLEN), jnp.float32),
            pltpu.VMEM((MAXLEN, BCH), jnp.int32),
            pltpu.VMEM((NBUF, BCH, EMBED_DIM), jnp.float32),
            pltpu.VMEM((NBUF, EMBED_DIM, PITCH), jnp.float32),
            pltpu.SemaphoreType.DMA,
            pltpu.SemaphoreType.DMA,
            pltpu.SemaphoreType.DMA,
            pltpu.SemaphoreType.DMA,
            pltpu.SemaphoreType.DMA,
            pltpu.SemaphoreType.DMA,
            pltpu.SemaphoreType.DMA,
            pltpu.SemaphoreType.DMA,
        ],
        compiler_params=pltpu.CompilerParams(use_tc_tiling_on_sc=False,
                                             needs_layout_passes=False),
    )
    def tok_pos_embed(x_hbm, tok_hbm, pos_hbm, out_hbm,
                      pos_v, idx_v, gbuf, tbuf,
                      g0, g1, g2, g3, o0, o1, o2, o3):
        wid = lax.axis_index("s") * NUM_CORES + lax.axis_index("c")
        b0 = wid * BCH
        gsem = (g0, g1, g2, g3)
        osem = (o0, o1, o2, o3)
        pltpu.sync_copy(pos_hbm, pos_v)
        pltpu.sync_copy(x_hbm.at[:, pl.ds(b0, BCH)], idx_v)

        def start_gather(l, bb):
            pltpu.async_copy(tok_hbm.at[idx_v.at[l]], gbuf.at[bb], gsem[bb])

        for bb in range(NBUF):
            start_gather(bb, bb)

        rows_dq = [lax.iota(jnp.int32, LANES) + dq * LANES for dq in range(DQ)]

        def group_body(g, carry):
            for bb in range(NBUF):
                l = g * NBUF + bb
                pltpu.make_async_copy(
                    tok_hbm.at[idx_v.at[l]], gbuf.at[bb], gsem[bb]).wait()

                @pl.when(g >= 1)
                def _wait_prev_out():
                    pltpu.make_async_copy(
                        tbuf.at[bb, :, pl.ds(0, BCH)],
                        out_hbm.at[0, :, pl.ds(b0, BCH)], osem[bb]).wait()

                l_splat = jnp.full((LANES,), l, jnp.int32)
                posc = [plsc.load_gather(pos_v, [rows_dq[dq], l_splat])
                        for dq in range(DQ)]

                def per_token(r, cr):
                    cols = jnp.full((LANES,), r, jnp.int32)
                    for dq in range(DQ):
                        v = gbuf[bb, r, pl.ds(dq * LANES, LANES)] + posc[dq]
                        plsc.store_scatter(tbuf.at[bb], [rows_dq[dq], cols], v)
                    return cr

                lax.fori_loop(0, BCH, per_token, 0, unroll=4)

                @pl.when(g < NGROUPS - 1)
                def _next_gather():
                    start_gather(l + NBUF, bb)

                pltpu.async_copy(
                    tbuf.at[bb, :, pl.ds(0, BCH)],
                    out_hbm.at[l, :, pl.ds(b0, BCH)], osem[bb])
            return carry

        lax.fori_loop(0, NGROUPS, group_body, 0)
        for bb in range(NBUF):
            pltpu.make_async_copy(
                tbuf.at[bb, :, pl.ds(0, BCH)],
                out_hbm.at[0, :, pl.ds(b0, BCH)], osem[bb]).wait()

    return tok_pos_embed


_kernel_call = _make_kernel()


def kernel(x, token_table, pos_table):
    x_t = jnp.transpose(x.astype(jnp.int32), (1, 0))
    pos_t = jnp.transpose(pos_table, (1, 0))
    out = _kernel_call(x_t, token_table, pos_t)
    return jnp.transpose(out, (2, 0, 1))


# speedup vs baseline: 1.9492x; 1.2247x over previous
import functools

import jax
import jax.numpy as jnp
from jax import lax
from jax.experimental import pallas as pl
from jax.experimental.pallas import tpu as pltpu
from jax.experimental.pallas import tpu_sc as plsc

VOCAB = 1000000
MAXLEN = 200
EMBED_DIM = 64
BATCH = 4096

NUM_CORES = 2
NUM_SUBCORES = 16
LANES = 16
NW = NUM_CORES * NUM_SUBCORES
BCH = BATCH // NW
DQ = EMBED_DIM // LANES
PITCH = BCH + 1
NBUF = 4
NGROUPS = MAXLEN // NBUF
LH = MAXLEN // 8


def _make_kernel():
    mesh = plsc.VectorSubcoreMesh(core_axis_name="c", subcore_axis_name="s")

    @functools.partial(
        pl.kernel,
        out_type=jax.ShapeDtypeStruct((MAXLEN, 8, NW, 8, BCH), jnp.float32),
        mesh=mesh,
        scratch_types=[
            pltpu.VMEM((EMBED_DIM, MAXLEN), jnp.float32),
            pltpu.VMEM((LH, 8, BCH), jnp.int32),
            pltpu.VMEM((NBUF, BCH, EMBED_DIM), jnp.float32),
            pltpu.VMEM((NBUF, 8, 8, PITCH), jnp.float32),
            pltpu.SemaphoreType.DMA,
            pltpu.SemaphoreType.DMA,
            pltpu.SemaphoreType.DMA,
            pltpu.SemaphoreType.DMA,
            pltpu.SemaphoreType.DMA,
            pltpu.SemaphoreType.DMA,
            pltpu.SemaphoreType.DMA,
            pltpu.SemaphoreType.DMA,
        ],
        compiler_params=pltpu.CompilerParams(use_tc_tiling_on_sc=False,
                                             needs_layout_passes=False),
    )
    def tok_pos_embed(x_hbm, tok_hbm, pos_hbm, out_hbm,
                      pos_v, idx_v, gbuf, tbuf,
                      g0, g1, g2, g3, o0, o1, o2, o3):
        wid = lax.axis_index("s") * NUM_CORES + lax.axis_index("c")
        gsem = (g0, g1, g2, g3)
        osem = (o0, o1, o2, o3)
        pltpu.sync_copy(pos_hbm, pos_v)
        pltpu.sync_copy(x_hbm.at[:, wid, :, :], idx_v)

        def start_gather(l, bb):
            pltpu.async_copy(tok_hbm.at[idx_v.at[l >> 3, l & 7]],
                             gbuf.at[bb], gsem[bb])

        for bb in range(NBUF):
            start_gather(bb, bb)

        rows_hi = [(lax.iota(jnp.int32, LANES) + dq * LANES) >> 3
                   for dq in range(DQ)]
        rows_lo = [(lax.iota(jnp.int32, LANES) + dq * LANES) & 7
                   for dq in range(DQ)]
        rows_dq = [lax.iota(jnp.int32, LANES) + dq * LANES for dq in range(DQ)]

        def group_body(g, carry):
            for bb in range(NBUF):
                l = g * NBUF + bb
                pltpu.make_async_copy(
                    tok_hbm.at[idx_v.at[l >> 3, l & 7]],
                    gbuf.at[bb], gsem[bb]).wait()

                @pl.when(g >= 1)
                def _wait_prev_out():
                    pltpu.make_async_copy(
                        tbuf.at[bb, :, :, pl.ds(0, BCH)],
                        out_hbm.at[0, :, wid, :, :], osem[bb]).wait()

                l_splat = jnp.full((LANES,), l, jnp.int32)
                posc = [plsc.load_gather(pos_v, [rows_dq[dq], l_splat])
                        for dq in range(DQ)]

                def per_token(r, cr):
                    cols = jnp.full((LANES,), r, jnp.int32)
                    for dq in range(DQ):
                        v = gbuf[bb, r, pl.ds(dq * LANES, LANES)] + posc[dq]
                        plsc.store_scatter(
                            tbuf.at[bb], [rows_hi[dq], rows_lo[dq], cols], v)
                    return cr

                lax.fori_loop(0, BCH, per_token, 0, unroll=4)

                @pl.when(g < NGROUPS - 1)
                def _next_gather():
                    start_gather(l + NBUF, bb)

                pltpu.async_copy(
                    tbuf.at[bb, :, :, pl.ds(0, BCH)],
                    out_hbm.at[l, :, wid, :, :], osem[bb])
            return carry

        lax.fori_loop(0, NGROUPS, group_body, 0)
        for bb in range(NBUF):
            pltpu.make_async_copy(
                tbuf.at[bb, :, :, pl.ds(0, BCH)],
                out_hbm.at[0, :, wid, :, :], osem[bb]).wait()

    return tok_pos_embed


_kernel_call = _make_kernel()


def kernel(x, token_table, pos_table):
    x4 = (x.astype(jnp.int32)
          .transpose(1, 0)
          .reshape(LH, 8, NW, BCH)
          .transpose(0, 2, 1, 3))
    pos_t = jnp.transpose(pos_table, (1, 0))
    out5 = _kernel_call(x4, token_table, pos_t)
    return (out5.transpose(2, 4, 0, 1, 3)
            .reshape(BATCH, MAXLEN, EMBED_DIM))
